# baseline (speedup 1.0000x reference)
import dataclasses
import functools

import jax
import jax.numpy as jnp
from jax import lax
from jax.experimental import pallas as pl
from jax.experimental.pallas import tpu as pltpu
from jax.experimental.pallas import tpu_sc as plsc

B = 4096
F = 26
D = 64
H0 = 200
H1 = 100

NC = 2
NS = 16
NW = NC * NS
BPW = B // NW
G = 4
GF = G * F
STEPS = BPW // G
EPW = BPW * F
DC = D // 16


def _sc_compiler_params():
    cp = pltpu.CompilerParams()
    fields = pltpu.CompilerParams.__dataclass_fields__
    if "needs_layout_passes" in fields:
        cp = dataclasses.replace(cp, needs_layout_passes=False)
    if "use_tc_tiling_on_sc" in fields:
        cp = dataclasses.replace(cp, use_tc_tiling_on_sc=False)
    return cp


def _sc_pool(idx_flat, vals_flat, emb_table, fmw_flat):
    mesh = plsc.VectorSubcoreMesh(core_axis_name="c", subcore_axis_name="s")

    @functools.partial(
        pl.kernel,
        compiler_params=_sc_compiler_params(),
        out_type=jax.ShapeDtypeStruct((B, 2 * D), jnp.float32),
        mesh=mesh,
        scratch_types=[
            pltpu.VMEM((EPW,), jnp.int32),
            pltpu.VMEM((EPW,), jnp.int32),
            pltpu.VMEM((EPW,), jnp.float32),
            pltpu.VMEM((EPW,), jnp.float32),
            pltpu.VMEM((2, GF, D), jnp.float32),
            pltpu.VMEM((BPW, 2 * D), jnp.float32),
            pltpu.SemaphoreType.DMA,
            pltpu.SemaphoreType.DMA,
            pltpu.SemaphoreType.DMA,
        ],
    )
    def sck(idx_hbm, vals_hbm, emb_hbm, fmw_hbm, fm2_hbm,
            idx_v, idx2_v, vals_v, fmw_v, rows_v, o2_v, sem0, sem1, semf):
        wid = lax.axis_index("s") * NC + lax.axis_index("c")
        base_e = pl.multiple_of(wid * EPW, 8)
        base_r = pl.multiple_of(wid * BPW, 8)

        pltpu.sync_copy(idx_hbm.at[pl.ds(base_e, EPW)], idx_v)
        pltpu.sync_copy(vals_hbm.at[pl.ds(base_e, EPW)], vals_v)

        halfv = 50000
        iota16i = lax.iota(jnp.int32, 16)

        @pl.loop(0, EPW // 16)
        def _(i):
            src16 = plsc.load_gather(idx_v, [i * 16 + iota16i])
            two = src16 + src16
            rem = jnp.where(src16 < halfv, two, two - (2 * halfv - 1))
            plsc.store_scatter(idx2_v, [i * 16 + iota16i], rem)

        sems = (sem0, sem1)

        def emb_desc(step, buf):
            off = pl.multiple_of(step * GF, 8)
            return pltpu.make_async_copy(
                emb_hbm.at[idx2_v.at[pl.ds(off, GF)]], rows_v.at[buf], sems[buf])

        def fm_desc(step):
            off = pl.multiple_of(step * GF, 8)
            return pltpu.make_async_copy(
                fmw_hbm.at[idx_v.at[pl.ds(off, GF)]],
                fmw_v.at[pl.ds(off, GF)], semf)

        emb_desc(0, 0).start()
        emb_desc(1, 1).start()

        def compute(step, buf):
            rows = rows_v.at[buf]

            @pl.loop(0, G)
            def _(g):
                r_local = step * G + g
                acc = [jnp.zeros((16,), jnp.float32) for _ in range(DC)]
                sq = [jnp.zeros((16,), jnp.float32) for _ in range(DC)]
                for f in range(F):
                    off16 = jnp.full((16,), r_local * F + f, jnp.int32)
                    val16 = plsc.load_gather(vals_v, [off16])
                    row = g * F + f
                    for c in range(DC):
                        e = rows[row, pl.ds(c * 16, 16)]
                        t = e * val16
                        acc[c] = acc[c] + t
                        sq[c] = sq[c] + t * t
                for c in range(DC):
                    o2_v[r_local, pl.ds(c * 16, 16)] = (
                        0.5 * (acc[c] * acc[c] - sq[c]))

        @pl.loop(0, STEPS, step=2)
        def _(s0):
            emb_desc(s0, 0).wait()
            compute(s0, 0)

            @pl.when(s0 + 2 < STEPS)
            def _():
                emb_desc(s0 + 2, 0).start()

            fm_desc(s0).start()
            emb_desc(s0 + 1, 1).wait()
            compute(s0 + 1, 1)

            @pl.when(s0 + 3 < STEPS)
            def _():
                emb_desc(s0 + 3, 1).start()

            fm_desc(s0 + 1).start()

        for s in range(STEPS):
            fm_desc(s).wait()

        iota16 = lax.iota(jnp.int32, 16)

        @pl.loop(0, BPW // 16)
        def _(rg):
            accw = jnp.zeros((16,), jnp.float32)
            for f in range(F):
                idx16 = (rg * 16 + iota16) * F + f
                w16 = plsc.load_gather(fmw_v, [idx16])
                v16 = plsc.load_gather(vals_v, [idx16])
                accw = accw + w16 * v16
            row16 = rg * 16 + iota16
            col16 = jnp.full((16,), D, jnp.int32)
            plsc.store_scatter(o2_v, [row16, col16], accw)

        pltpu.sync_copy(o2_v, fm2_hbm.at[pl.ds(base_r, BPW)])

    return sck(idx_flat, vals_flat, emb_table, fmw_flat)


PACK_ROWS = 1000


def _pack_body(x_ref, out_ref):
    j = pl.program_id(1)

    @pl.when(j == 0)
    def _():
        out_ref[:, :D] = x_ref[...]

    @pl.when(j == 1)
    def _():
        out_ref[:, D:] = x_ref[...]


def _pack2(emb_table):
    v = emb_table.shape[0]
    half_blocks = (v // 2) // PACK_ROWS
    return pl.pallas_call(
        _pack_body,
        grid=(half_blocks, 2),
        in_specs=[pl.BlockSpec((PACK_ROWS, D),
                               lambda i, j, hb=half_blocks: (i + j * hb, 0))],
        out_specs=pl.BlockSpec((PACK_ROWS, 2 * D), lambda i, j: (i, 0)),
        out_shape=jax.ShapeDtypeStruct((v // 2, 2 * D), jnp.float32),
    )(emb_table)


BLK = 1024


def _mlp_body(fm2_ref, w0_ref, b0_ref, w1_ref, b1_ref, fcw_ref,
              c0_ref, out_ref):
    xa = fm2_ref[...]
    x = xa[:, :D]
    fm1 = xa[:, D:D + 1]
    h = jnp.dot(x, w0_ref[...], preferred_element_type=jnp.float32,
                precision=lax.Precision.DEFAULT) + b0_ref[...]
    h = jnp.maximum(h, 0.0)
    h = jnp.dot(h, w1_ref[...], preferred_element_type=jnp.float32,
                precision=lax.Precision.DEFAULT) + b1_ref[...]
    h = jnp.maximum(h, 0.0)
    logit = jnp.sum(h * fcw_ref[...], axis=1, keepdims=True)
    out_ref[...] = jax.nn.sigmoid(logit + fm1 + c0_ref[...])


def _mlp(fm2, W0, b0_row, W1, b1_row, fcw_row, c0):
    return pl.pallas_call(
        _mlp_body,
        grid=(B // BLK,),
        in_specs=[
            pl.BlockSpec((BLK, 2 * D), lambda i: (i, 0)),
            pl.BlockSpec((D, H0), lambda i: (0, 0)),
            pl.BlockSpec((1, H0), lambda i: (0, 0)),
            pl.BlockSpec((H0, H1), lambda i: (0, 0)),
            pl.BlockSpec((1, H1), lambda i: (0, 0)),
            pl.BlockSpec((1, H1), lambda i: (0, 0)),
            pl.BlockSpec((1, 1), lambda i: (0, 0)),
        ],
        out_specs=pl.BlockSpec((BLK, 1), lambda i: (i, 0)),
        out_shape=jax.ShapeDtypeStruct((B, 1), jnp.float32),
    )(fm2, W0, b0_row, W1, b1_row, fcw_row, c0)


def kernel(idxs, vals, emb_table, fm_w, W0, b0, W1, b1, fcW, fcb, bias):
    idx_flat = idxs.reshape(-1)
    vals_flat = vals.reshape(-1)
    fmw_flat = fm_w.reshape(-1)
    emb_lin = _pack2(emb_table).reshape(emb_table.shape)
    fm2 = _sc_pool(idx_flat, vals_flat, emb_lin, fmw_flat)
    c0 = (fcb[0] + bias[0]).reshape(1, 1)
    out = _mlp(fm2, W0, b0.reshape(1, H0), W1,
               b1.reshape(1, H1), fcW.reshape(1, H1), c0)
    return out.reshape(B)


# speedup vs baseline: 1.4828x; 1.4828x over previous
import dataclasses
import functools

import jax
import jax.numpy as jnp
from jax import lax
from jax.experimental import pallas as pl
from jax.experimental.pallas import tpu as pltpu
from jax.experimental.pallas import tpu_sc as plsc

B = 4096
F = 26
D = 64
H0 = 200
H1 = 100

NC = 2
NS = 16
NW = NC * NS
BPW = B // NW
G = 4
GF = G * F
STEPS = BPW // G
EPW = BPW * F
DC = D // 16


def _sc_compiler_params():
    cp = pltpu.CompilerParams()
    fields = pltpu.CompilerParams.__dataclass_fields__
    if "needs_layout_passes" in fields:
        cp = dataclasses.replace(cp, needs_layout_passes=False)
    if "use_tc_tiling_on_sc" in fields:
        cp = dataclasses.replace(cp, use_tc_tiling_on_sc=False)
    return cp


def _sc_pool(idx_flat, vals_flat, emb_table, fmw_flat):
    mesh = plsc.VectorSubcoreMesh(core_axis_name="c", subcore_axis_name="s")

    @functools.partial(
        pl.kernel,
        compiler_params=_sc_compiler_params(),
        out_type=jax.ShapeDtypeStruct((B, 2 * D), jnp.float32),
        mesh=mesh,
        scratch_types=[
            pltpu.VMEM((EPW,), jnp.int32),
            pltpu.VMEM((EPW,), jnp.float32),
            pltpu.VMEM((EPW,), jnp.float32),
            pltpu.VMEM((2, GF, D), jnp.float32),
            pltpu.VMEM((BPW, 2 * D), jnp.float32),
            pltpu.VMEM_SHARED((100000,), jnp.float32),
            pltpu.SemaphoreType.DMA,
            pltpu.SemaphoreType.DMA,
            pltpu.SemaphoreType.DMA,
            pltpu.SemaphoreType.DMA,
        ],
    )
    def sck(idx_hbm, vals_hbm, emb_hbm, fmw_hbm, fm2_hbm,
            idx_v, vals_v, fmw_v, rows_v, o2_v, fmw_sh, sem0, sem1, semf,
            semsh):
        sid = lax.axis_index("s")
        wid = sid * NC + lax.axis_index("c")

        fill = pltpu.make_async_copy(fmw_hbm, fmw_sh, semsh)

        @pl.when(sid == 0)
        def _():
            fill.start()
        base_e = pl.multiple_of(wid * EPW, 8)
        base_r = pl.multiple_of(wid * BPW, 8)

        pltpu.sync_copy(idx_hbm.at[pl.ds(base_e, EPW)], idx_v)
        pltpu.sync_copy(vals_hbm.at[pl.ds(base_e, EPW)], vals_v)

        sems = (sem0, sem1)

        def emb_desc(step, buf):
            off = pl.multiple_of(step * GF, 8)
            return pltpu.make_async_copy(
                emb_hbm.at[idx_v.at[pl.ds(off, GF)]], rows_v.at[buf], sems[buf])

        def fm_desc(step):
            off = pl.multiple_of(step * GF, 8)
            return pltpu.make_async_copy(
                fmw_sh.at[idx_v.at[pl.ds(off, GF)]],
                fmw_v.at[pl.ds(off, GF)], semf)

        emb_desc(0, 0).start()
        emb_desc(1, 1).start()

        @pl.when(sid == 0)
        def _():
            fill.wait()

        plsc.subcore_barrier()

        def compute(step, buf):
            rows = rows_v.at[buf]

            @pl.loop(0, G)
            def _(g):
                r_local = step * G + g
                acc = [jnp.zeros((16,), jnp.float32) for _ in range(DC)]
                sq = [jnp.zeros((16,), jnp.float32) for _ in range(DC)]
                for f in range(F):
                    off16 = jnp.full((16,), r_local * F + f, jnp.int32)
                    val16 = plsc.load_gather(vals_v, [off16])
                    row = g * F + f
                    for c in range(DC):
                        e = rows[row, pl.ds(c * 16, 16)]
                        t = e * val16
                        acc[c] = acc[c] + t
                        sq[c] = sq[c] + t * t
                for c in range(DC):
                    o2_v[r_local, pl.ds(c * 16, 16)] = (
                        0.5 * (acc[c] * acc[c] - sq[c]))

        @pl.loop(0, STEPS, step=2)
        def _(s0):
            emb_desc(s0, 0).wait()
            compute(s0, 0)

            @pl.when(s0 + 2 < STEPS)
            def _():
                emb_desc(s0 + 2, 0).start()

            emb_desc(s0 + 1, 1).wait()
            compute(s0 + 1, 1)

            @pl.when(s0 + 3 < STEPS)
            def _():
                emb_desc(s0 + 3, 1).start()

        for s in range(STEPS):
            fm_desc(s).start()
        for s in range(STEPS):
            fm_desc(s).wait()

        iota16 = lax.iota(jnp.int32, 16)

        @pl.loop(0, BPW // 16)
        def _(rg):
            accw = jnp.zeros((16,), jnp.float32)
            for f in range(F):
                idx16 = (rg * 16 + iota16) * F + f
                w16 = plsc.load_gather(fmw_v, [idx16])
                v16 = plsc.load_gather(vals_v, [idx16])
                accw = accw + w16 * v16
            row16 = rg * 16 + iota16
            col16 = jnp.full((16,), D, jnp.int32)
            plsc.store_scatter(o2_v, [row16, col16], accw)

        pltpu.sync_copy(o2_v, fm2_hbm.at[pl.ds(base_r, BPW)])

    return sck(idx_flat, vals_flat, emb_table, fmw_flat)


BLK = 1024


def _mlp_body(fm2_ref, w0_ref, b0_ref, w1_ref, b1_ref, fcw_ref,
              c0_ref, out_ref):
    xa = fm2_ref[...]
    x = xa[:, :D]
    fm1 = xa[:, D:D + 1]
    h = jnp.dot(x, w0_ref[...], preferred_element_type=jnp.float32,
                precision=lax.Precision.DEFAULT) + b0_ref[...]
    h = jnp.maximum(h, 0.0)
    h = jnp.dot(h, w1_ref[...], preferred_element_type=jnp.float32,
                precision=lax.Precision.DEFAULT) + b1_ref[...]
    h = jnp.maximum(h, 0.0)
    logit = jnp.sum(h * fcw_ref[...], axis=1, keepdims=True)
    out_ref[...] = jax.nn.sigmoid(logit + fm1 + c0_ref[...])


def _mlp(fm2, W0, b0_row, W1, b1_row, fcw_row, c0):
    return pl.pallas_call(
        _mlp_body,
        grid=(B // BLK,),
        in_specs=[
            pl.BlockSpec((BLK, 2 * D), lambda i: (i, 0)),
            pl.BlockSpec((D, H0), lambda i: (0, 0)),
            pl.BlockSpec((1, H0), lambda i: (0, 0)),
            pl.BlockSpec((H0, H1), lambda i: (0, 0)),
            pl.BlockSpec((1, H1), lambda i: (0, 0)),
            pl.BlockSpec((1, H1), lambda i: (0, 0)),
            pl.BlockSpec((1, 1), lambda i: (0, 0)),
        ],
        out_specs=pl.BlockSpec((BLK, 1), lambda i: (i, 0)),
        out_shape=jax.ShapeDtypeStruct((B, 1), jnp.float32),
    )(fm2, W0, b0_row, W1, b1_row, fcw_row, c0)


def kernel(idxs, vals, emb_table, fm_w, W0, b0, W1, b1, fcW, fcb, bias):
    idx_flat = idxs.reshape(-1)
    vals_flat = vals.reshape(-1)
    fmw_flat = fm_w.reshape(-1)
    fm2 = _sc_pool(idx_flat, vals_flat, emb_table, fmw_flat)
    c0 = (fcb[0] + bias[0]).reshape(1, 1)
    out = _mlp(fm2, W0, b0.reshape(1, H0), W1,
               b1.reshape(1, H1), fcW.reshape(1, H1), c0)
    return out.reshape(B)
